# baseline (speedup 1.0000x reference)
import jax
import jax.numpy as jnp
from jax import lax
from jax.experimental import pallas as pl
from jax.experimental.pallas import tpu as pltpu

N = 10000
E = 160000
D = 256
DE = 16
Z2 = 2 * D

NB = 2000
EB = 2000


def _proj_node_body(x_ref, wg_ref, wh_ref, bg_ref, g_ref, h_ref):
    xb = x_ref[...]
    g_ref[...] = (jnp.dot(xb, wg_ref[...], preferred_element_type=jnp.float32)
                  + bg_ref[...])
    h_ref[...] = jnp.dot(xb, wh_ref[...], preferred_element_type=jnp.float32)


def _proj_edge_body(ea_ref, we_ref, c_ref):
    c_ref[...] = jnp.dot(ea_ref[...], we_ref[...],
                         preferred_element_type=jnp.float32)


def _edge_act_body(z_ref, m_ref):
    z = z_ref[...]
    zf = z[:, :D]
    zs = z[:, D:]
    gate = 1.0 / (1.0 + jnp.exp(-zf))
    sp = jnp.maximum(zs, 0.0) + jnp.log1p(jnp.exp(-jnp.abs(zs)))
    m_ref[...] = gate * sp


def _stats_body(agg_ref, x_ref, out_ref, acc_ref):
    n = pl.program_id(0)
    conv = agg_ref[...] + x_ref[...]
    st = jnp.concatenate([jnp.sum(conv, axis=0, keepdims=True),
                          jnp.sum(conv * conv, axis=0, keepdims=True)], axis=0)

    @pl.when(n == 0)
    def _():
        acc_ref[...] = st

    @pl.when(n > 0)
    def _():
        acc_ref[...] += st

    @pl.when(n == pl.num_programs(0) - 1)
    def _():
        out_ref[...] = acc_ref[...]


def _final_body(agg_ref, x_ref, st_ref, gm_ref, bt_ref, out_ref):
    xb = x_ref[...]
    conv = agg_ref[...] + xb
    inv_n = jnp.float32(1.0 / N)
    mean = st_ref[0:1, :] * inv_n
    var = st_ref[1:2, :] * inv_n - mean * mean
    rstd = lax.rsqrt(var + jnp.float32(1e-5))
    normed = (conv - mean) * rstd * gm_ref[...] + bt_ref[...]
    act = normed * (1.0 / (1.0 + jnp.exp(-normed)))
    out_ref[...] = xb + act


def kernel(x, edge_index, edge_attr, Wf, bf, Ws, bs, gamma, beta):
    f32 = jnp.float32
    src = edge_index[0]
    dst = edge_index[1]

    Wg = jnp.concatenate([Wf[:D], Ws[:D]], axis=1)
    Wh = jnp.concatenate([Wf[D:2 * D], Ws[D:2 * D]], axis=1)
    We = jnp.concatenate([Wf[2 * D:], Ws[2 * D:]], axis=1)
    bg = jnp.concatenate([bf, bs]).reshape(1, Z2)

    nb = N // NB
    g, h = pl.pallas_call(
        _proj_node_body,
        grid=(nb,),
        in_specs=[
            pl.BlockSpec((NB, D), lambda n: (n, 0)),
            pl.BlockSpec((D, Z2), lambda n: (0, 0)),
            pl.BlockSpec((D, Z2), lambda n: (0, 0)),
            pl.BlockSpec((1, Z2), lambda n: (0, 0)),
        ],
        out_specs=[pl.BlockSpec((NB, Z2), lambda n: (n, 0)),
                   pl.BlockSpec((NB, Z2), lambda n: (n, 0))],
        out_shape=[jax.ShapeDtypeStruct((N, Z2), f32),
                   jax.ShapeDtypeStruct((N, Z2), f32)],
    )(x, Wg, Wh, bg)

    eb = E // EB
    c = pl.pallas_call(
        _proj_edge_body,
        grid=(eb,),
        in_specs=[
            pl.BlockSpec((EB, DE), lambda e: (e, 0)),
            pl.BlockSpec((DE, Z2), lambda e: (0, 0)),
        ],
        out_specs=pl.BlockSpec((EB, Z2), lambda e: (e, 0)),
        out_shape=jax.ShapeDtypeStruct((E, Z2), f32),
    )(edge_attr, We)

    z = g[dst] + h[src] + c

    m = pl.pallas_call(
        _edge_act_body,
        grid=(eb,),
        in_specs=[pl.BlockSpec((EB, Z2), lambda e: (e, 0))],
        out_specs=pl.BlockSpec((EB, D), lambda e: (e, 0)),
        out_shape=jax.ShapeDtypeStruct((E, D), f32),
    )(z)

    agg = jax.ops.segment_sum(m, dst, num_segments=N)

    stats = pl.pallas_call(
        _stats_body,
        grid=(nb,),
        in_specs=[pl.BlockSpec((NB, D), lambda n: (n, 0)),
                  pl.BlockSpec((NB, D), lambda n: (n, 0))],
        out_specs=pl.BlockSpec((2, D), lambda n: (0, 0)),
        out_shape=jax.ShapeDtypeStruct((2, D), f32),
        scratch_shapes=[pltpu.VMEM((2, D), f32)],
    )(agg, x)

    out = pl.pallas_call(
        _final_body,
        grid=(nb,),
        in_specs=[
            pl.BlockSpec((NB, D), lambda n: (n, 0)),
            pl.BlockSpec((NB, D), lambda n: (n, 0)),
            pl.BlockSpec((2, D), lambda n: (0, 0)),
            pl.BlockSpec((1, D), lambda n: (0, 0)),
            pl.BlockSpec((1, D), lambda n: (0, 0)),
        ],
        out_specs=pl.BlockSpec((NB, D), lambda n: (n, 0)),
        out_shape=jax.ShapeDtypeStruct((N, D), f32),
    )(agg, x, stats, gamma.reshape(1, D), beta.reshape(1, D))

    return out


# speedup vs baseline: 1.0607x; 1.0607x over previous
import jax
import jax.numpy as jnp
from jax import lax
from jax.experimental import pallas as pl
from jax.experimental.pallas import tpu as pltpu

N = 10000
E = 160000
D = 256
DE = 16
Z2 = 2 * D

NB = 2000
EB = 2000


def _proj_node_body(x_ref, wg_ref, wh_ref, bg_ref, g_ref, h_ref):
    xb = x_ref[...]
    g_ref[...] = (jnp.dot(xb, wg_ref[...], preferred_element_type=jnp.float32)
                  + bg_ref[...]).astype(jnp.bfloat16)
    h_ref[...] = jnp.dot(xb, wh_ref[...],
                         preferred_element_type=jnp.float32
                         ).astype(jnp.bfloat16)


def _proj_edge_body(ea_ref, we_ref, c_ref):
    c_ref[...] = jnp.dot(ea_ref[...], we_ref[...],
                         preferred_element_type=jnp.float32)


def _edge_act_body(z_ref, m_ref):
    z = z_ref[...]
    zf = z[:, :D]
    zs = z[:, D:]
    gate = 1.0 / (1.0 + jnp.exp(-zf))
    sp = jnp.maximum(zs, 0.0) + jnp.log1p(jnp.exp(-jnp.abs(zs)))
    m_ref[...] = gate * sp


def _stats_body(agg_ref, x_ref, out_ref, acc_ref):
    n = pl.program_id(0)
    conv = agg_ref[...] + x_ref[...]
    st = jnp.concatenate([jnp.sum(conv, axis=0, keepdims=True),
                          jnp.sum(conv * conv, axis=0, keepdims=True)], axis=0)

    @pl.when(n == 0)
    def _():
        acc_ref[...] = st

    @pl.when(n > 0)
    def _():
        acc_ref[...] += st

    @pl.when(n == pl.num_programs(0) - 1)
    def _():
        out_ref[...] = acc_ref[...]


def _final_body(agg_ref, x_ref, st_ref, gm_ref, bt_ref, out_ref):
    xb = x_ref[...]
    conv = agg_ref[...] + xb
    inv_n = jnp.float32(1.0 / N)
    mean = st_ref[0:1, :] * inv_n
    var = st_ref[1:2, :] * inv_n - mean * mean
    rstd = lax.rsqrt(var + jnp.float32(1e-5))
    normed = (conv - mean) * rstd * gm_ref[...] + bt_ref[...]
    act = normed * (1.0 / (1.0 + jnp.exp(-normed)))
    out_ref[...] = xb + act


def kernel(x, edge_index, edge_attr, Wf, bf, Ws, bs, gamma, beta):
    f32 = jnp.float32
    src = edge_index[0]
    dst = edge_index[1]

    Wg = jnp.concatenate([Wf[:D], Ws[:D]], axis=1)
    Wh = jnp.concatenate([Wf[D:2 * D], Ws[D:2 * D]], axis=1)
    We = jnp.concatenate([Wf[2 * D:], Ws[2 * D:]], axis=1)
    bg = jnp.concatenate([bf, bs]).reshape(1, Z2)

    nb = N // NB
    g, h = pl.pallas_call(
        _proj_node_body,
        grid=(nb,),
        in_specs=[
            pl.BlockSpec((NB, D), lambda n: (n, 0)),
            pl.BlockSpec((D, Z2), lambda n: (0, 0)),
            pl.BlockSpec((D, Z2), lambda n: (0, 0)),
            pl.BlockSpec((1, Z2), lambda n: (0, 0)),
        ],
        out_specs=[pl.BlockSpec((NB, Z2), lambda n: (n, 0)),
                   pl.BlockSpec((NB, Z2), lambda n: (n, 0))],
        out_shape=[jax.ShapeDtypeStruct((N, Z2), jnp.bfloat16),
                   jax.ShapeDtypeStruct((N, Z2), jnp.bfloat16)],
    )(x, Wg, Wh, bg)

    eb = E // EB
    c = pl.pallas_call(
        _proj_edge_body,
        grid=(eb,),
        in_specs=[
            pl.BlockSpec((EB, DE), lambda e: (e, 0)),
            pl.BlockSpec((DE, Z2), lambda e: (0, 0)),
        ],
        out_specs=pl.BlockSpec((EB, Z2), lambda e: (e, 0)),
        out_shape=jax.ShapeDtypeStruct((E, Z2), f32),
    )(edge_attr, We)

    z = g[dst].astype(f32) + h[src].astype(f32) + c

    m = pl.pallas_call(
        _edge_act_body,
        grid=(eb,),
        in_specs=[pl.BlockSpec((EB, Z2), lambda e: (e, 0))],
        out_specs=pl.BlockSpec((EB, D), lambda e: (e, 0)),
        out_shape=jax.ShapeDtypeStruct((E, D), f32),
    )(z)

    agg = jax.ops.segment_sum(m, dst, num_segments=N)

    stats = pl.pallas_call(
        _stats_body,
        grid=(nb,),
        in_specs=[pl.BlockSpec((NB, D), lambda n: (n, 0)),
                  pl.BlockSpec((NB, D), lambda n: (n, 0))],
        out_specs=pl.BlockSpec((2, D), lambda n: (0, 0)),
        out_shape=jax.ShapeDtypeStruct((2, D), f32),
        scratch_shapes=[pltpu.VMEM((2, D), f32)],
    )(agg, x)

    out = pl.pallas_call(
        _final_body,
        grid=(nb,),
        in_specs=[
            pl.BlockSpec((NB, D), lambda n: (n, 0)),
            pl.BlockSpec((NB, D), lambda n: (n, 0)),
            pl.BlockSpec((2, D), lambda n: (0, 0)),
            pl.BlockSpec((1, D), lambda n: (0, 0)),
            pl.BlockSpec((1, D), lambda n: (0, 0)),
        ],
        out_specs=pl.BlockSpec((NB, D), lambda n: (n, 0)),
        out_shape=jax.ShapeDtypeStruct((N, D), f32),
    )(agg, x, stats, gamma.reshape(1, D), beta.reshape(1, D))

    return out
